# Initial kernel scaffold; baseline (speedup 1.0000x reference)
import jax
import jax.numpy as jnp
from jax.experimental import pallas as pl


def kernel(x, edge_index, W1_rel, b1_rel, W1_root, W2_rel, b2_rel, W2_root, W3_rel, b3_rel, W3_root):
    raise NotImplementedError("write your pallas kernel here")


# speedup vs baseline: 9.1340x; 9.1340x over previous
import functools

import jax
import jax.numpy as jnp
from jax import lax
from jax.experimental import pallas as pl
from jax.experimental.pallas import tpu as pltpu
from jax.experimental.pallas import tpu_sc as plsc

_N = 10000
_E = 320000
_D = 128
_H = 128

_NC = 2
_NS = 16
_NW = _NC * _NS
_EPW = _E // _NW
_C = 80
_NCHUNK = _EPW // _C
_W = 25
_NWIN = _NCHUNK // _W
_RPT = 640
_RPT_LAST = _N - 15 * _RPT
_DEGP = 10240
_DPT = _DEGP // _NS


def _seg_body(h_hbm, zero_hbm, src_hbm, dst_hbm, agg_out,
              src_v, dst_v, rows_v, acc_sh, gsem):
    c = lax.axis_index("c")
    s = lax.axis_index("s")
    wid = s * _NC + c

    @pl.when(s < 15)
    def _():
        pltpu.sync_copy(zero_hbm.at[pl.ds(s * _RPT, _RPT)],
                        acc_sh.at[pl.ds(s * _RPT, _RPT)])

    @pl.when(s == 15)
    def _():
        pltpu.sync_copy(zero_hbm.at[pl.ds(15 * _RPT, _RPT_LAST)],
                        acc_sh.at[pl.ds(15 * _RPT, _RPT_LAST)])

    plsc.subcore_barrier()

    def win(w, carry):
        pltpu.sync_copy(src_hbm.at[wid, w], src_v)
        pltpu.sync_copy(dst_hbm.at[wid, w], dst_v)

        def step(j, carry2):
            pltpu.async_copy(h_hbm.at[src_v.at[j]], rows_v, gsem).wait()
            pltpu.sync_copy(rows_v, acc_sh.at[dst_v.at[j]], add=True)
            return carry2

        return lax.fori_loop(0, _W, step, carry)

    lax.fori_loop(0, _NWIN, win, 0)
    plsc.subcore_barrier()

    @pl.when(s < 15)
    def _():
        pltpu.sync_copy(acc_sh.at[pl.ds(s * _RPT, _RPT)],
                        agg_out.at[c, pl.ds(s * _RPT, _RPT)])

    @pl.when(s == 15)
    def _():
        pltpu.sync_copy(acc_sh.at[pl.ds(15 * _RPT, _RPT_LAST)],
                        agg_out.at[c, pl.ds(15 * _RPT, _RPT_LAST)])


def _make_segsum():
    mesh = plsc.VectorSubcoreMesh(core_axis_name="c", subcore_axis_name="s")
    return pl.kernel(
        _seg_body,
        out_type=[jax.ShapeDtypeStruct((_NC, _N, _H), jnp.float32)],
        mesh=mesh,
        scratch_types=[
            pltpu.VMEM((_W, _C), jnp.int32),
            pltpu.VMEM((_W, _C), jnp.int32),
            pltpu.VMEM((_C, _H), jnp.float32),
            pltpu.VMEM_SHARED((_N, _H), jnp.float32),
            pltpu.SemaphoreType.DMA,
        ],
    )


_segsum = _make_segsum()


def _deg_body(ones_hbm, zero_hbm, src_hbm, deg_out,
              src_v, ones_v, deg_sh, gsem):
    c = lax.axis_index("c")
    s = lax.axis_index("s")
    wid = s * _NC + c

    pltpu.sync_copy(zero_hbm.at[pl.ds(s * _DPT, _DPT)],
                    deg_sh.at[pl.ds(s * _DPT, _DPT)])
    pltpu.sync_copy(ones_hbm, ones_v)
    plsc.subcore_barrier()

    def win(w, carry):
        pltpu.sync_copy(src_hbm.at[wid, w], src_v)

        def step(j, carry2):
            pltpu.sync_copy(ones_v, deg_sh.at[src_v.at[j]], add=True)
            return carry2

        return lax.fori_loop(0, _W, step, carry)

    lax.fori_loop(0, _NWIN, win, 0)
    plsc.subcore_barrier()

    pltpu.sync_copy(deg_sh.at[pl.ds(s * _DPT, _DPT)],
                    deg_out.at[c, pl.ds(s * _DPT, _DPT)])


def _make_deg():
    mesh = plsc.VectorSubcoreMesh(core_axis_name="c", subcore_axis_name="s")
    return pl.kernel(
        _deg_body,
        out_type=[jax.ShapeDtypeStruct((_NC, _DEGP, _H), jnp.float32)],
        mesh=mesh,
        scratch_types=[
            pltpu.VMEM((_W, _C), jnp.int32),
            pltpu.VMEM((_C, _H), jnp.float32),
            pltpu.VMEM_SHARED((_DEGP, _H), jnp.float32),
            pltpu.SemaphoreType.DMA,
        ],
    )


_deghist = _make_deg()


def _bdot(a, b):
    return jnp.dot(a.astype(jnp.bfloat16), b.astype(jnp.bfloat16),
                   preferred_element_type=jnp.float32)


def _layer_body(a_ref, h_ref, wr_ref, wo_ref, b_ref, o_ref):
    agg = a_ref[0] + a_ref[1]
    o_ref[...] = jnp.maximum(
        _bdot(agg, wr_ref[...]) + _bdot(h_ref[...], wo_ref[...])
        + b_ref[...], 0.0)


def _final_body(a_ref, h_ref, deg_ref, wr_ref, wo_ref, b_ref,
                w3r_ref, w3o_ref, b3_ref, out_ref, sall_acc, sdeg_acc):
    i = pl.program_id(0)

    @pl.when(i == 0)
    def _():
        sall_acc[...] = jnp.zeros_like(sall_acc)
        sdeg_acc[...] = jnp.zeros_like(sdeg_acc)

    agg = a_ref[0] + a_ref[1]
    h2 = jnp.maximum(
        _bdot(agg, wr_ref[...]) + _bdot(h_ref[...], wo_ref[...])
        + b_ref[...], 0.0)
    sall_acc[...] += jnp.sum(h2, axis=0, keepdims=True)
    sdeg_acc[...] += jnp.sum(deg_ref[...] * h2, axis=0, keepdims=True)

    @pl.when(i == pl.num_programs(0) - 1)
    def _():
        w3r = w3r_ref[...].astype(jnp.bfloat16).astype(jnp.float32)
        w3o = w3o_ref[...].astype(jnp.bfloat16).astype(jnp.float32)
        out_ref[...] = (
            jnp.sum(sdeg_acc[...] * w3r, axis=1, keepdims=True) / _N
            + jnp.sum(sall_acc[...] * w3o, axis=1, keepdims=True) / _N
            + b3_ref[...])


_BLK_R = 2000


def _layer(a, h, wr_t, wo_t, b):
    return pl.pallas_call(
        _layer_body,
        grid=(_N // _BLK_R,),
        in_specs=[
            pl.BlockSpec((_NC, _BLK_R, _H), lambda i: (0, i, 0)),
            pl.BlockSpec((_BLK_R, _H), lambda i: (i, 0)),
            pl.BlockSpec((_H, _H), lambda i: (0, 0)),
            pl.BlockSpec((_H, _H), lambda i: (0, 0)),
            pl.BlockSpec((1, _H), lambda i: (0, 0)),
        ],
        out_specs=pl.BlockSpec((_BLK_R, _H), lambda i: (i, 0)),
        out_shape=jax.ShapeDtypeStruct((_N, _H), jnp.float32),
    )(a, h, wr_t, wo_t, b)


def _final(a, h, degv, wr_t, wo_t, b, w3r, w3o, b3):
    return pl.pallas_call(
        _final_body,
        grid=(_N // _BLK_R,),
        in_specs=[
            pl.BlockSpec((_NC, _BLK_R, _H), lambda i: (0, i, 0)),
            pl.BlockSpec((_BLK_R, _H), lambda i: (i, 0)),
            pl.BlockSpec((_BLK_R, 1), lambda i: (i, 0)),
            pl.BlockSpec((_H, _H), lambda i: (0, 0)),
            pl.BlockSpec((_H, _H), lambda i: (0, 0)),
            pl.BlockSpec((1, _H), lambda i: (0, 0)),
            pl.BlockSpec((1, _H), lambda i: (0, 0)),
            pl.BlockSpec((1, _H), lambda i: (0, 0)),
            pl.BlockSpec((1, 1), lambda i: (0, 0)),
        ],
        out_specs=pl.BlockSpec((1, 1), lambda i: (0, 0)),
        out_shape=jax.ShapeDtypeStruct((1, 1), jnp.float32),
        scratch_shapes=[
            pltpu.VMEM((1, _H), jnp.float32),
            pltpu.VMEM((1, _H), jnp.float32),
        ],
    )(a, h, degv, wr_t, wo_t, b, w3r, w3o, b3)


def kernel(x, edge_index, W1_rel, b1_rel, W1_root, W2_rel, b2_rel, W2_root,
           W3_rel, b3_rel, W3_root):
    src = edge_index[0].reshape(_NW, _NWIN, _W, _C)
    dst = edge_index[1].reshape(_NW, _NWIN, _W, _C)
    zero = jnp.zeros((_N, _H), jnp.float32)

    ones = jnp.ones((_C, _H), jnp.float32)
    zdeg = jnp.zeros((_DEGP, _H), jnp.float32)

    (agg1,) = _segsum(x, zero, src, dst)
    (deg,) = _deghist(ones, zdeg, src)
    h1 = _layer(agg1, x, W1_rel.T, W1_root.T, b1_rel.reshape(1, _H))
    (agg2,) = _segsum(h1, zero, src, dst)
    degv = deg[0, :_N, :1] + deg[1, :_N, :1]
    return _final(agg2, h1, degv, W2_rel.T, W2_root.T, b2_rel.reshape(1, _H),
                  W3_rel, W3_root, b3_rel.reshape(1, 1))


# speedup vs baseline: 11.0143x; 1.2059x over previous
import functools

import jax
import jax.numpy as jnp
from jax import lax
from jax.experimental import pallas as pl
from jax.experimental.pallas import tpu as pltpu
from jax.experimental.pallas import tpu_sc as plsc

_N = 10000
_E = 320000
_D = 128
_H = 128

_NC = 2
_NS = 16
_NW = _NC * _NS
_EPW = _E // _NW
_C = 80
_NCHUNK = _EPW // _C
_W = 25
_NWIN = _NCHUNK // _W
_RPT = 640
_RPT_LAST = _N - 15 * _RPT
_DEGP = 10240
_DPT = _DEGP // _NS


def _seg_body(h_hbm, zero_hbm, src_hbm, dst_hbm, agg_out,
              src_v, dst_v, rows_v, rows_v2, acc_sh,
              gsem, gsem2, ssem, ssem2):
    c = lax.axis_index("c")
    s = lax.axis_index("s")
    wid = s * _NC + c

    @pl.when(s < 15)
    def _():
        pltpu.sync_copy(zero_hbm.at[pl.ds(s * _RPT, _RPT)],
                        acc_sh.at[pl.ds(s * _RPT, _RPT)])

    @pl.when(s == 15)
    def _():
        pltpu.sync_copy(zero_hbm.at[pl.ds(15 * _RPT, _RPT_LAST)],
                        acc_sh.at[pl.ds(15 * _RPT, _RPT_LAST)])

    plsc.subcore_barrier()

    rows = (rows_v, rows_v2)
    gsems = (gsem, gsem2)
    ssems = (ssem, ssem2)

    def win(w, carry):
        pltpu.sync_copy(src_hbm.at[wid, w], src_v)
        pltpu.sync_copy(dst_hbm.at[wid, w], dst_v)

        gd = [None, None]
        sd = [None, None]
        gd[0] = pltpu.async_copy(h_hbm.at[src_v.at[0]], rows[0], gsems[0])
        for j in range(_W):
            b = j % 2
            nb = (j + 1) % 2
            gd[b].wait()
            sd[b] = pltpu.async_copy(rows[b], acc_sh.at[dst_v.at[j]],
                                     ssems[b], add=True)
            if j + 1 < _W:
                if sd[nb] is not None:
                    sd[nb].wait()
                gd[nb] = pltpu.async_copy(h_hbm.at[src_v.at[j + 1]],
                                          rows[nb], gsems[nb])
        sd[(_W - 1) % 2].wait()
        sd[_W % 2].wait()
        return carry

    lax.fori_loop(0, _NWIN, win, 0)
    plsc.subcore_barrier()

    @pl.when(s < 15)
    def _():
        pltpu.sync_copy(acc_sh.at[pl.ds(s * _RPT, _RPT)],
                        agg_out.at[c, pl.ds(s * _RPT, _RPT)])

    @pl.when(s == 15)
    def _():
        pltpu.sync_copy(acc_sh.at[pl.ds(15 * _RPT, _RPT_LAST)],
                        agg_out.at[c, pl.ds(15 * _RPT, _RPT_LAST)])


def _make_segsum():
    mesh = plsc.VectorSubcoreMesh(core_axis_name="c", subcore_axis_name="s")
    return pl.kernel(
        _seg_body,
        out_type=[jax.ShapeDtypeStruct((_NC, _N, _H), jnp.float32)],
        mesh=mesh,
        scratch_types=[
            pltpu.VMEM((_W, _C), jnp.int32),
            pltpu.VMEM((_W, _C), jnp.int32),
            pltpu.VMEM((_C, _H), jnp.float32),
            pltpu.VMEM((_C, _H), jnp.float32),
            pltpu.VMEM_SHARED((_N, _H), jnp.float32),
            pltpu.SemaphoreType.DMA,
            pltpu.SemaphoreType.DMA,
            pltpu.SemaphoreType.DMA,
            pltpu.SemaphoreType.DMA,
        ],
    )


_segsum = _make_segsum()


def _deg_body(ones_hbm, zero_hbm, src_hbm, deg_out,
              src_v, ones_v, deg_sh, gsem):
    c = lax.axis_index("c")
    s = lax.axis_index("s")
    wid = s * _NC + c

    pltpu.sync_copy(zero_hbm.at[pl.ds(s * _DPT, _DPT)],
                    deg_sh.at[pl.ds(s * _DPT, _DPT)])
    pltpu.sync_copy(ones_hbm, ones_v)
    plsc.subcore_barrier()

    def win(w, carry):
        pltpu.sync_copy(src_hbm.at[wid, w], src_v)
        ds = [pltpu.async_copy(ones_v, deg_sh.at[src_v.at[j]], gsem,
                               add=True) for j in range(_W)]
        for d in ds:
            d.wait()
        return carry

    lax.fori_loop(0, _NWIN, win, 0)
    plsc.subcore_barrier()

    pltpu.sync_copy(deg_sh.at[pl.ds(s * _DPT, _DPT)],
                    deg_out.at[c, pl.ds(s * _DPT, _DPT)])


def _make_deg():
    mesh = plsc.VectorSubcoreMesh(core_axis_name="c", subcore_axis_name="s")
    return pl.kernel(
        _deg_body,
        out_type=[jax.ShapeDtypeStruct((_NC, _DEGP, _H), jnp.float32)],
        mesh=mesh,
        scratch_types=[
            pltpu.VMEM((_W, _C), jnp.int32),
            pltpu.VMEM((_C, _H), jnp.float32),
            pltpu.VMEM_SHARED((_DEGP, _H), jnp.float32),
            pltpu.SemaphoreType.DMA,
        ],
    )


_deghist = _make_deg()


def _bdot(a, b):
    return jnp.dot(a.astype(jnp.bfloat16), b.astype(jnp.bfloat16),
                   preferred_element_type=jnp.float32)


def _layer_body(a_ref, h_ref, wr_ref, wo_ref, b_ref, o_ref):
    agg = a_ref[0] + a_ref[1]
    o_ref[...] = jnp.maximum(
        _bdot(agg, wr_ref[...]) + _bdot(h_ref[...], wo_ref[...])
        + b_ref[...], 0.0)


def _final_body(a_ref, h_ref, deg_ref, wr_ref, wo_ref, b_ref,
                w3r_ref, w3o_ref, b3_ref, out_ref, sall_acc, sdeg_acc):
    i = pl.program_id(0)

    @pl.when(i == 0)
    def _():
        sall_acc[...] = jnp.zeros_like(sall_acc)
        sdeg_acc[...] = jnp.zeros_like(sdeg_acc)

    agg = a_ref[0] + a_ref[1]
    h2 = jnp.maximum(
        _bdot(agg, wr_ref[...]) + _bdot(h_ref[...], wo_ref[...])
        + b_ref[...], 0.0)
    sall_acc[...] += jnp.sum(h2, axis=0, keepdims=True)
    sdeg_acc[...] += jnp.sum(deg_ref[...] * h2, axis=0, keepdims=True)

    @pl.when(i == pl.num_programs(0) - 1)
    def _():
        w3r = w3r_ref[...].astype(jnp.bfloat16).astype(jnp.float32)
        w3o = w3o_ref[...].astype(jnp.bfloat16).astype(jnp.float32)
        out_ref[...] = (
            jnp.sum(sdeg_acc[...] * w3r, axis=1, keepdims=True) / _N
            + jnp.sum(sall_acc[...] * w3o, axis=1, keepdims=True) / _N
            + b3_ref[...])


_BLK_R = 2000


def _layer(a, h, wr_t, wo_t, b):
    return pl.pallas_call(
        _layer_body,
        grid=(_N // _BLK_R,),
        in_specs=[
            pl.BlockSpec((_NC, _BLK_R, _H), lambda i: (0, i, 0)),
            pl.BlockSpec((_BLK_R, _H), lambda i: (i, 0)),
            pl.BlockSpec((_H, _H), lambda i: (0, 0)),
            pl.BlockSpec((_H, _H), lambda i: (0, 0)),
            pl.BlockSpec((1, _H), lambda i: (0, 0)),
        ],
        out_specs=pl.BlockSpec((_BLK_R, _H), lambda i: (i, 0)),
        out_shape=jax.ShapeDtypeStruct((_N, _H), jnp.float32),
    )(a, h, wr_t, wo_t, b)


def _final(a, h, degv, wr_t, wo_t, b, w3r, w3o, b3):
    return pl.pallas_call(
        _final_body,
        grid=(_N // _BLK_R,),
        in_specs=[
            pl.BlockSpec((_NC, _BLK_R, _H), lambda i: (0, i, 0)),
            pl.BlockSpec((_BLK_R, _H), lambda i: (i, 0)),
            pl.BlockSpec((_BLK_R, 1), lambda i: (i, 0)),
            pl.BlockSpec((_H, _H), lambda i: (0, 0)),
            pl.BlockSpec((_H, _H), lambda i: (0, 0)),
            pl.BlockSpec((1, _H), lambda i: (0, 0)),
            pl.BlockSpec((1, _H), lambda i: (0, 0)),
            pl.BlockSpec((1, _H), lambda i: (0, 0)),
            pl.BlockSpec((1, 1), lambda i: (0, 0)),
        ],
        out_specs=pl.BlockSpec((1, 1), lambda i: (0, 0)),
        out_shape=jax.ShapeDtypeStruct((1, 1), jnp.float32),
        scratch_shapes=[
            pltpu.VMEM((1, _H), jnp.float32),
            pltpu.VMEM((1, _H), jnp.float32),
        ],
    )(a, h, degv, wr_t, wo_t, b, w3r, w3o, b3)


def kernel(x, edge_index, W1_rel, b1_rel, W1_root, W2_rel, b2_rel, W2_root,
           W3_rel, b3_rel, W3_root):
    src = edge_index[0].reshape(_NW, _NWIN, _W, _C)
    dst = edge_index[1].reshape(_NW, _NWIN, _W, _C)
    zero = jnp.zeros((_N, _H), jnp.float32)

    ones = jnp.ones((_C, _H), jnp.float32)
    zdeg = jnp.zeros((_DEGP, _H), jnp.float32)

    (agg1,) = _segsum(x, zero, src, dst)
    (deg,) = _deghist(ones, zdeg, src)
    h1 = _layer(agg1, x, W1_rel.T, W1_root.T, b1_rel.reshape(1, _H))
    (agg2,) = _segsum(h1, zero, src, dst)
    degv = deg[0, :_N, :1] + deg[1, :_N, :1]
    return _final(agg2, h1, degv, W2_rel.T, W2_root.T, b2_rel.reshape(1, _H),
                  W3_rel, W3_root, b3_rel.reshape(1, 1))


# speedup vs baseline: 11.7806x; 1.0696x over previous
import functools

import jax
import jax.numpy as jnp
from jax import lax
from jax.experimental import pallas as pl
from jax.experimental.pallas import tpu as pltpu
from jax.experimental.pallas import tpu_sc as plsc

_N = 10000
_E = 320000
_D = 128
_H = 128

_NC = 2
_NS = 16
_NW = _NC * _NS
_EPW = _E // _NW
_C = 100
_NCHUNK = _EPW // _C
_W = 20
_NWIN = _NCHUNK // _W
_RPT = 640
_RPT_LAST = _N - 15 * _RPT
_DEGP = 10240
_DPT = _DEGP // _NS


def _seg_body(h_hbm, zero_hbm, src_hbm, dst_hbm, agg_out,
              src_v, dst_v, rows_v, rows_v2, acc_sh,
              gsem, gsem2, ssem, ssem2):
    c = lax.axis_index("c")
    s = lax.axis_index("s")
    wid = s * _NC + c

    @pl.when(s < 15)
    def _():
        pltpu.sync_copy(zero_hbm.at[pl.ds(s * _RPT, _RPT)],
                        acc_sh.at[pl.ds(s * _RPT, _RPT)])

    @pl.when(s == 15)
    def _():
        pltpu.sync_copy(zero_hbm.at[pl.ds(15 * _RPT, _RPT_LAST)],
                        acc_sh.at[pl.ds(15 * _RPT, _RPT_LAST)])

    plsc.subcore_barrier()

    rows = (rows_v, rows_v2)
    gsems = (gsem, gsem2)
    ssems = (ssem, ssem2)

    def win(w, carry):
        pltpu.sync_copy(src_hbm.at[wid, w], src_v)
        pltpu.sync_copy(dst_hbm.at[wid, w], dst_v)

        gd = [None, None]
        sd = [None, None]
        gd[0] = pltpu.async_copy(h_hbm.at[src_v.at[0]], rows[0], gsems[0])
        for j in range(_W):
            b = j % 2
            nb = (j + 1) % 2
            gd[b].wait()
            sd[b] = pltpu.async_copy(rows[b], acc_sh.at[dst_v.at[j]],
                                     ssems[b], add=True)
            if j + 1 < _W:
                if sd[nb] is not None:
                    sd[nb].wait()
                gd[nb] = pltpu.async_copy(h_hbm.at[src_v.at[j + 1]],
                                          rows[nb], gsems[nb])
        sd[(_W - 1) % 2].wait()
        sd[_W % 2].wait()
        return carry

    lax.fori_loop(0, _NWIN, win, 0)
    plsc.subcore_barrier()

    @pl.when(s < 15)
    def _():
        pltpu.sync_copy(acc_sh.at[pl.ds(s * _RPT, _RPT)],
                        agg_out.at[c, pl.ds(s * _RPT, _RPT)])

    @pl.when(s == 15)
    def _():
        pltpu.sync_copy(acc_sh.at[pl.ds(15 * _RPT, _RPT_LAST)],
                        agg_out.at[c, pl.ds(15 * _RPT, _RPT_LAST)])


def _make_segsum():
    mesh = plsc.VectorSubcoreMesh(core_axis_name="c", subcore_axis_name="s")
    return pl.kernel(
        _seg_body,
        out_type=[jax.ShapeDtypeStruct((_NC, _N, _H), jnp.float32)],
        mesh=mesh,
        scratch_types=[
            pltpu.VMEM((_W, _C), jnp.int32),
            pltpu.VMEM((_W, _C), jnp.int32),
            pltpu.VMEM((_C, _H), jnp.float32),
            pltpu.VMEM((_C, _H), jnp.float32),
            pltpu.VMEM_SHARED((_N, _H), jnp.float32),
            pltpu.SemaphoreType.DMA,
            pltpu.SemaphoreType.DMA,
            pltpu.SemaphoreType.DMA,
            pltpu.SemaphoreType.DMA,
        ],
    )


_segsum = _make_segsum()


def _deg_body(ones_hbm, zero_hbm, src_hbm, deg_out,
              src_v, ones_v, deg_sh, gsem):
    c = lax.axis_index("c")
    s = lax.axis_index("s")
    wid = s * _NC + c

    pltpu.sync_copy(zero_hbm.at[pl.ds(s * _DPT, _DPT)],
                    deg_sh.at[pl.ds(s * _DPT, _DPT)])
    pltpu.sync_copy(ones_hbm, ones_v)
    plsc.subcore_barrier()

    def win(w, carry):
        pltpu.sync_copy(src_hbm.at[wid, w], src_v)
        ds = [pltpu.async_copy(ones_v, deg_sh.at[src_v.at[j]], gsem,
                               add=True) for j in range(_W)]
        for d in ds:
            d.wait()
        return carry

    lax.fori_loop(0, _NWIN, win, 0)
    plsc.subcore_barrier()

    pltpu.sync_copy(deg_sh.at[pl.ds(s * _DPT, _DPT)],
                    deg_out.at[c, pl.ds(s * _DPT, _DPT)])


def _make_deg():
    mesh = plsc.VectorSubcoreMesh(core_axis_name="c", subcore_axis_name="s")
    return pl.kernel(
        _deg_body,
        out_type=[jax.ShapeDtypeStruct((_NC, _DEGP, _H), jnp.float32)],
        mesh=mesh,
        scratch_types=[
            pltpu.VMEM((_W, _C), jnp.int32),
            pltpu.VMEM((_C, _H), jnp.float32),
            pltpu.VMEM_SHARED((_DEGP, _H), jnp.float32),
            pltpu.SemaphoreType.DMA,
        ],
    )


_deghist = _make_deg()


def _bdot(a, b):
    return jnp.dot(a.astype(jnp.bfloat16), b.astype(jnp.bfloat16),
                   preferred_element_type=jnp.float32)


def _layer_body(a_ref, h_ref, wr_ref, wo_ref, b_ref, o_ref):
    agg = a_ref[0] + a_ref[1]
    o_ref[...] = jnp.maximum(
        _bdot(agg, wr_ref[...]) + _bdot(h_ref[...], wo_ref[...])
        + b_ref[...], 0.0)


def _final_body(a_ref, h_ref, deg_ref, wr_ref, wo_ref, b_ref,
                w3r_ref, w3o_ref, b3_ref, out_ref, sall_acc, sdeg_acc):
    i = pl.program_id(0)

    @pl.when(i == 0)
    def _():
        sall_acc[...] = jnp.zeros_like(sall_acc)
        sdeg_acc[...] = jnp.zeros_like(sdeg_acc)

    agg = a_ref[0] + a_ref[1]
    h2 = jnp.maximum(
        _bdot(agg, wr_ref[...]) + _bdot(h_ref[...], wo_ref[...])
        + b_ref[...], 0.0)
    sall_acc[...] += jnp.sum(h2, axis=0, keepdims=True)
    sdeg_acc[...] += jnp.sum(deg_ref[...] * h2, axis=0, keepdims=True)

    @pl.when(i == pl.num_programs(0) - 1)
    def _():
        w3r = w3r_ref[...].astype(jnp.bfloat16).astype(jnp.float32)
        w3o = w3o_ref[...].astype(jnp.bfloat16).astype(jnp.float32)
        out_ref[...] = (
            jnp.sum(sdeg_acc[...] * w3r, axis=1, keepdims=True) / _N
            + jnp.sum(sall_acc[...] * w3o, axis=1, keepdims=True) / _N
            + b3_ref[...])


_BLK_R = 2000


def _layer(a, h, wr_t, wo_t, b):
    return pl.pallas_call(
        _layer_body,
        grid=(_N // _BLK_R,),
        in_specs=[
            pl.BlockSpec((_NC, _BLK_R, _H), lambda i: (0, i, 0)),
            pl.BlockSpec((_BLK_R, _H), lambda i: (i, 0)),
            pl.BlockSpec((_H, _H), lambda i: (0, 0)),
            pl.BlockSpec((_H, _H), lambda i: (0, 0)),
            pl.BlockSpec((1, _H), lambda i: (0, 0)),
        ],
        out_specs=pl.BlockSpec((_BLK_R, _H), lambda i: (i, 0)),
        out_shape=jax.ShapeDtypeStruct((_N, _H), jnp.float32),
    )(a, h, wr_t, wo_t, b)


def _final(a, h, degv, wr_t, wo_t, b, w3r, w3o, b3):
    return pl.pallas_call(
        _final_body,
        grid=(_N // _BLK_R,),
        in_specs=[
            pl.BlockSpec((_NC, _BLK_R, _H), lambda i: (0, i, 0)),
            pl.BlockSpec((_BLK_R, _H), lambda i: (i, 0)),
            pl.BlockSpec((_BLK_R, 1), lambda i: (i, 0)),
            pl.BlockSpec((_H, _H), lambda i: (0, 0)),
            pl.BlockSpec((_H, _H), lambda i: (0, 0)),
            pl.BlockSpec((1, _H), lambda i: (0, 0)),
            pl.BlockSpec((1, _H), lambda i: (0, 0)),
            pl.BlockSpec((1, _H), lambda i: (0, 0)),
            pl.BlockSpec((1, 1), lambda i: (0, 0)),
        ],
        out_specs=pl.BlockSpec((1, 1), lambda i: (0, 0)),
        out_shape=jax.ShapeDtypeStruct((1, 1), jnp.float32),
        scratch_shapes=[
            pltpu.VMEM((1, _H), jnp.float32),
            pltpu.VMEM((1, _H), jnp.float32),
        ],
    )(a, h, degv, wr_t, wo_t, b, w3r, w3o, b3)


def kernel(x, edge_index, W1_rel, b1_rel, W1_root, W2_rel, b2_rel, W2_root,
           W3_rel, b3_rel, W3_root):
    src = edge_index[0].reshape(_NW, _NWIN, _W, _C)
    dst = edge_index[1].reshape(_NW, _NWIN, _W, _C)
    zero = jnp.zeros((_N, _H), jnp.float32)

    ones = jnp.ones((_C, _H), jnp.float32)
    zdeg = jnp.zeros((_DEGP, _H), jnp.float32)

    (agg1,) = _segsum(x, zero, src, dst)
    (deg,) = _deghist(ones, zdeg, src)
    h1 = _layer(agg1, x, W1_rel.T, W1_root.T, b1_rel.reshape(1, _H))
    (agg2,) = _segsum(h1, zero, src, dst)
    degv = deg[0, :_N, :1] + deg[1, :_N, :1]
    return _final(agg2, h1, degv, W2_rel.T, W2_root.T, b2_rel.reshape(1, _H),
                  W3_rel, W3_root, b3_rel.reshape(1, 1))
